# Initial kernel scaffold; baseline (speedup 1.0000x reference)
import functools

import jax
import jax.numpy as jnp
from jax.experimental import pallas as pl
from jax.experimental.pallas import tpu as pltpu


def _ce_body(x_ref, out_ref, *, nj):
    j = pl.program_id(1)

    @pl.when(j == 0)
    def _init():
        out_ref[...] = jnp.zeros_like(out_ref)

    x = x_ref[...]
    r = pltpu.roll(x, -1, 1)
    d = x - r
    sp = jnp.maximum(d, 0.0) + jnp.log1p(jnp.exp(-jnp.abs(d)))
    lane = jax.lax.broadcasted_iota(jnp.int32, x.shape, 1)
    sp = jnp.where(lane % 2 == 0, sp, 0.0)
    out_ref[...] = out_ref[...] + jnp.sum(sp)


def kernel(logits):
    B, C = logits.shape
    total = B * C

    W = 512
    R = total // W
    flat = logits.reshape(R, W)

    BR = 1024
    nj = R // (2 * BR)
    grid = (2, nj)

    partials = pl.pallas_call(
        functools.partial(_ce_body, nj=nj),
        out_shape=jax.ShapeDtypeStruct((2, 1, 1), jnp.float32),
        grid=grid,
        in_specs=[pl.BlockSpec((BR, W), lambda i, j: (i * nj + j, 0))],
        out_specs=pl.BlockSpec((1, 1, 1), lambda i, j: (i, 0, 0)),
        compiler_params=pltpu.CompilerParams(
            dimension_semantics=("parallel", "arbitrary"),
        ),
    )(flat)
    return partials.sum() * (1.0 / B)


# speedup vs baseline: 1.3892x; 1.3892x over previous
import functools

import jax
import jax.numpy as jnp
from jax.experimental import pallas as pl
from jax.experimental.pallas import tpu as pltpu


def _ce_body(x_ref, out_ref, *, nj):
    j = pl.program_id(1)

    @pl.when(j == 0)
    def _init():
        out_ref[...] = jnp.zeros_like(out_ref)

    x = x_ref[...]
    r = pltpu.roll(x, x.shape[1] - 1, 1)
    d = x - r
    sp = jnp.maximum(d, 0.0) + jnp.log1p(jnp.exp(-jnp.abs(d)))
    lane = jax.lax.broadcasted_iota(jnp.int32, x.shape, 1)
    sp = jnp.where(lane % 2 == 0, sp, 0.0)
    out_ref[...] = out_ref[...] + jnp.sum(sp)


def kernel(logits):
    B, C = logits.shape
    total = B * C

    W = 512
    R = total // W
    flat = logits.reshape(R, W)

    BR = 1024
    nj = R // (2 * BR)
    grid = (2, nj)

    partials = pl.pallas_call(
        functools.partial(_ce_body, nj=nj),
        out_shape=jax.ShapeDtypeStruct((2, 1, 1), jnp.float32),
        grid=grid,
        in_specs=[pl.BlockSpec((BR, W), lambda i, j: (i * nj + j, 0))],
        out_specs=pl.BlockSpec((1, 1, 1), lambda i, j: (i, 0, 0)),
        compiler_params=pltpu.CompilerParams(
            dimension_semantics=("parallel", "arbitrary"),
        ),
    )(flat)
    return partials.sum() * (1.0 / B)


# speedup vs baseline: 3.4046x; 2.4508x over previous
import functools

import jax
import jax.numpy as jnp
from jax.experimental import pallas as pl
from jax.experimental.pallas import tpu as pltpu


def _ce_body(x_ref, out_ref, *, nj):
    j = pl.program_id(1)

    @pl.when(j == 0)
    def _init():
        out_ref[...] = jnp.zeros_like(out_ref)

    x = x_ref[...]
    d = x[:, :1] - x[:, 1:2]
    sp = jnp.maximum(d, 0.0) + jnp.log1p(jnp.exp(-jnp.abs(d)))
    out_ref[...] = out_ref[...] + jnp.sum(sp)


def kernel(logits):
    B, C = logits.shape

    TB = 16384
    nj = B // (2 * TB)
    grid = (2, nj)

    partials = pl.pallas_call(
        functools.partial(_ce_body, nj=nj),
        out_shape=jax.ShapeDtypeStruct((2, 1, 1), jnp.float32),
        grid=grid,
        in_specs=[pl.BlockSpec((TB, C), lambda i, j: (i * nj + j, 0))],
        out_specs=pl.BlockSpec((1, 1, 1), lambda i, j: (i, 0, 0)),
        compiler_params=pltpu.CompilerParams(
            dimension_semantics=("parallel", "arbitrary"),
        ),
    )(logits)
    return partials.sum() * (1.0 / B)


# speedup vs baseline: 3.4403x; 1.0105x over previous
import functools

import jax
import jax.numpy as jnp
from jax.experimental import pallas as pl
from jax.experimental.pallas import tpu as pltpu


def _ce_body(*refs, nj):
    x_refs, out_ref = refs[:-1], refs[-1]
    j = pl.program_id(1)

    @pl.when(j == 0)
    def _init():
        out_ref[...] = jnp.zeros_like(out_ref)

    acc = jnp.zeros((), jnp.float32)
    for x_ref in x_refs:
        x = x_ref[...]
        d = x[:, :1] - x[:, 1:2]
        sp = jnp.maximum(d, 0.0) + jnp.log1p(jnp.exp(-jnp.abs(d)))
        acc = acc + jnp.sum(sp)
    out_ref[...] = out_ref[...] + acc


def kernel(logits):
    B, C = logits.shape

    S = 4
    TB = 8192
    nj = B // (2 * S * TB)
    grid = (2, nj)

    def _spec(s):
        return pl.BlockSpec((TB, C), lambda i, j, s=s: ((i * S + s) * nj + j, 0))

    partials = pl.pallas_call(
        functools.partial(_ce_body, nj=nj),
        out_shape=jax.ShapeDtypeStruct((2, 1, 1), jnp.float32),
        grid=grid,
        in_specs=[_spec(s) for s in range(S)],
        out_specs=pl.BlockSpec((1, 1, 1), lambda i, j: (i, 0, 0)),
        compiler_params=pltpu.CompilerParams(
            dimension_semantics=("parallel", "arbitrary"),
        ),
    )(*([logits] * S))
    return partials.sum() * (1.0 / B)
